# baseline (speedup 1.0000x reference)
import functools

import jax
import jax.numpy as jnp
from jax import lax
from jax.experimental import pallas as pl
from jax.experimental.pallas import tpu as pltpu
from jax.experimental.pallas import tpu_sc as plsc

N = 1_000_000
K = 16_384
L = 16
NC, NS = 2, 16
NW = NC * NS
SHARD = 31_256
LAST_BASE = (NW - 1) * SHARD
LAST_LEN = N - LAST_BASE
CHUNKS = K // L
ROW = 2048
KSLICE = K // NS


_mesh = plsc.VectorSubcoreMesh(core_axis_name="c", subcore_axis_name="s")


@functools.partial(
    pl.kernel,
    mesh=_mesh,
    out_type=jax.ShapeDtypeStruct((N,), jnp.float32),
    scratch_types=[
        pltpu.VMEM((SHARD,), jnp.float32),
        pltpu.VMEM((K,), jnp.int32),
        pltpu.VMEM((K,), jnp.float32),
        pltpu.VMEM((ROW,), jnp.int32),
        pltpu.VMEM((ROW,), jnp.float32),
        pltpu.VMEM_SHARED((K,), jnp.int32),
        pltpu.VMEM_SHARED((K,), jnp.float32),
        pltpu.SemaphoreType.DMA,
        pltpu.SemaphoreType.DMA,
    ],
    compiler_params=pltpu.CompilerParams(needs_layout_passes=False),
)
def _scatter_set(in_hbm, idx_hbm, val_hbm, out_hbm, shard_v, idx_v, val_v,
                 cidx_v, cval_v, idx_sh, val_sh, sem, sem2):
    cid = lax.axis_index("c")
    sid = lax.axis_index("s")
    wid = sid * NC + cid
    base = wid * SHARD
    my_len = jnp.where(wid == NW - 1, LAST_LEN, SHARD)
    is_last = wid == NW - 1

    @pl.when(is_last)
    def _copy_last():
        pltpu.async_copy(in_hbm.at[pl.ds(LAST_BASE, LAST_LEN)],
                         shard_v.at[pl.ds(0, LAST_LEN)], sem)

    @pl.when(jnp.logical_not(is_last))
    def _copy_full():
        pltpu.async_copy(in_hbm.at[pl.ds(base, SHARD)], shard_v, sem)

    def wait_in_copy():
        @pl.when(is_last)
        def _wl():
            pltpu.make_async_copy(in_hbm.at[pl.ds(LAST_BASE, LAST_LEN)],
                                  shard_v.at[pl.ds(0, LAST_LEN)], sem).wait()

        @pl.when(jnp.logical_not(is_last))
        def _wf():
            pltpu.make_async_copy(in_hbm.at[pl.ds(base, SHARD)], shard_v,
                                  sem).wait()

    off = sid * KSLICE
    pltpu.sync_copy(idx_hbm.at[pl.ds(off, KSLICE)], idx_sh.at[pl.ds(off, KSLICE)])
    pltpu.sync_copy(val_hbm.at[pl.ds(off, KSLICE)], val_sh.at[pl.ds(off, KSLICE)])
    plsc.subcore_barrier()
    icpy = pltpu.async_copy(idx_sh, idx_v, sem2)
    vcpy = pltpu.async_copy(val_sh, val_v, sem2)
    icpy.wait()
    vcpy.wait()

    lane = lax.broadcasted_iota(jnp.int32, (L,), 0)

    def apply_rows(hi, n_valid):
        def abody(a, carry):
            sa = a * L
            valid = lane + sa < n_valid
            ci = cidx_v[pl.ds(sa, L)]
            cv = cval_v[pl.ds(sa, L)]
            relc = ci - base
            _, last2 = plsc.scan_count(relc, valid)
            plsc.store_scatter(shard_v, [relc], cv, mask=last2 & valid)
            return carry
        lax.fori_loop(0, hi, abody, 0)

    def body(c, carry):
        ptr, waited = carry
        s = c * L
        iv = idx_v[pl.ds(s, L)]
        vv = val_v[pl.ds(s, L)]
        rel = iv - base
        m = plsc.bitcast(rel, jnp.uint32) < my_len.astype(jnp.uint32)
        _, last = plsc.scan_count(rel, m)
        w = last & m
        cnt = jnp.sum(w.astype(jnp.int32))

        flush = ptr > ROW - L

        @pl.when(flush)
        def _flush():
            @pl.when(waited == 0)
            def _w():
                wait_in_copy()
            apply_rows(ROW // L, ptr)

        ptr = jnp.where(flush, 0, ptr)
        waited = jnp.where(flush, 1, waited)
        plsc.store_compressed(cidx_v.at[pl.ds(ptr, L)], iv, mask=w)
        plsc.store_compressed(cval_v.at[pl.ds(ptr, L)], vv, mask=w)
        return ptr + cnt, waited

    ptr, waited = lax.fori_loop(0, CHUNKS, body, (0, 0), unroll=8)

    @pl.when(waited == 0)
    def _wait_tail():
        wait_in_copy()

    apply_rows((ptr + L - 1) // L, ptr)

    @pl.when(is_last)
    def _out_last():
        pltpu.sync_copy(shard_v.at[pl.ds(0, LAST_LEN)],
                        out_hbm.at[pl.ds(LAST_BASE, LAST_LEN)])

    @pl.when(jnp.logical_not(is_last))
    def _out_full():
        pltpu.sync_copy(shard_v, out_hbm.at[pl.ds(base, SHARD)])


def kernel(input, index, value):
    return _scatter_set(input, index.astype(jnp.int32), value)


# speedup vs baseline: 1.4056x; 1.4056x over previous
import functools

import jax
import jax.numpy as jnp
from jax import lax
from jax.experimental import pallas as pl
from jax.experimental.pallas import tpu as pltpu
from jax.experimental.pallas import tpu_sc as plsc

N = 1_000_000
K = 16_384
L = 16
NC, NS = 2, 16
NW = NC * NS
SHARD = 31_256
LAST_BASE = N - SHARD
CHUNKS = K // L


_mesh = plsc.VectorSubcoreMesh(core_axis_name="c", subcore_axis_name="s")


@functools.partial(
    pl.kernel,
    mesh=_mesh,
    out_type=jax.ShapeDtypeStruct((N,), jnp.float32),
    scratch_types=[
        pltpu.VMEM((SHARD,), jnp.float32),
        pltpu.VMEM((K,), jnp.int32),
        pltpu.VMEM((K,), jnp.float32),
        pltpu.VMEM_SHARED((K,), jnp.int32),
        pltpu.VMEM_SHARED((K,), jnp.float32),
        pltpu.SemaphoreType.DMA,
        pltpu.SemaphoreType.DMA,
    ],
    compiler_params=pltpu.CompilerParams(needs_layout_passes=False),
)
def _scatter_set(in_hbm, idx_hbm, val_hbm, out_hbm, shard_v, idx_v, val_v,
                 idx_sh, val_sh, sem, sem2):
    cid = lax.axis_index("c")
    sid = lax.axis_index("s")
    wid = sid * NC + cid
    base = jnp.where(wid == NW - 1, LAST_BASE, wid * SHARD)

    shard_cpy = pltpu.async_copy(in_hbm.at[pl.ds(base, SHARD)], shard_v, sem)

    kslice = K // NS
    off = sid * kslice
    sicpy = pltpu.async_copy(idx_hbm.at[pl.ds(off, kslice)],
                             idx_sh.at[pl.ds(off, kslice)], sem2)
    svcpy = pltpu.async_copy(val_hbm.at[pl.ds(off, kslice)],
                             val_sh.at[pl.ds(off, kslice)], sem2)
    sicpy.wait()
    svcpy.wait()
    plsc.subcore_barrier()
    icpy = pltpu.async_copy(idx_sh, idx_v, sem2)
    vcpy = pltpu.async_copy(val_sh, val_v, sem2)
    icpy.wait()
    vcpy.wait()
    shard_cpy.wait()

    def body(c, carry):
        s = c * L
        iv = idx_v[pl.ds(s, L)]
        vv = val_v[pl.ds(s, L)]
        rel = iv - base
        m = plsc.bitcast(rel, jnp.uint32) < jnp.uint32(SHARD)
        _, last = plsc.scan_count(rel, m)
        plsc.store_scatter(shard_v, [rel], vv, mask=last & m)
        return carry

    lax.fori_loop(0, CHUNKS, body, 0, unroll=16)

    pltpu.sync_copy(shard_v, out_hbm.at[pl.ds(base, SHARD)])


def kernel(input, index, value):
    return _scatter_set(input, index.astype(jnp.int32), value)


# speedup vs baseline: 2.2088x; 1.5715x over previous
import functools

import jax
import jax.numpy as jnp
from jax import lax
from jax.experimental import pallas as pl
from jax.experimental.pallas import tpu as pltpu
from jax.experimental.pallas import tpu_sc as plsc

N = 1_000_000
K = 16_384
L = 16
NC, NS = 2, 16
NW = NC * NS
SHARD = 31_256
LAST_BASE = N - SHARD
CHUNKS = K // L


_mesh = plsc.VectorSubcoreMesh(core_axis_name="c", subcore_axis_name="s")


@functools.partial(
    pl.kernel,
    mesh=_mesh,
    out_type=jax.ShapeDtypeStruct((N,), jnp.float32),
    scratch_types=[
        pltpu.VMEM((SHARD,), jnp.float32),
        pltpu.VMEM((K,), jnp.int32),
        pltpu.VMEM((K,), jnp.float32),
        pltpu.VMEM_SHARED((K,), jnp.int32),
        pltpu.VMEM_SHARED((K,), jnp.float32),
        pltpu.SemaphoreType.DMA,
        pltpu.SemaphoreType.DMA,
    ],
    compiler_params=pltpu.CompilerParams(needs_layout_passes=False),
)
def _scatter_set(in_hbm, idx_hbm, val_hbm, out_hbm, shard_v, idx_v, val_v,
                 idx_sh, val_sh, sem, sem2):
    cid = lax.axis_index("c")
    sid = lax.axis_index("s")
    wid = sid * NC + cid
    base = jnp.where(wid == NW - 1, LAST_BASE, wid * SHARD)

    shard_cpy = pltpu.async_copy(in_hbm.at[pl.ds(base, SHARD)], shard_v, sem)

    kslice = K // NS
    off = sid * kslice
    sicpy = pltpu.async_copy(idx_hbm.at[pl.ds(off, kslice)],
                             idx_sh.at[pl.ds(off, kslice)], sem2)
    svcpy = pltpu.async_copy(val_hbm.at[pl.ds(off, kslice)],
                             val_sh.at[pl.ds(off, kslice)], sem2)
    sicpy.wait()
    svcpy.wait()
    plsc.subcore_barrier()
    icpy = pltpu.async_copy(idx_sh, idx_v, sem2)
    vcpy = pltpu.async_copy(val_sh, val_v, sem2)
    icpy.wait()
    vcpy.wait()
    shard_cpy.wait()

    U = 8

    def body(g, carry):
        s0 = g * (L * U)
        rels, vvs, ms = [], [], []
        for k in range(U):
            iv = idx_v[pl.ds(s0 + k * L, L)]
            vvs.append(val_v[pl.ds(s0 + k * L, L)])
            rel = iv - base
            rels.append(rel)
            ms.append(plsc.bitcast(rel, jnp.uint32) < jnp.uint32(SHARD))
        lasts = [plsc.scan_count(rels[k], ms[k])[1] for k in range(U)]
        for k in range(U):
            plsc.store_scatter(shard_v, [rels[k]], vvs[k],
                               mask=lasts[k] & ms[k])
        return carry

    lax.fori_loop(0, CHUNKS // U, body, 0, unroll=2)

    pltpu.sync_copy(shard_v, out_hbm.at[pl.ds(base, SHARD)])


def kernel(input, index, value):
    return _scatter_set(input, index.astype(jnp.int32), value)


# speedup vs baseline: 2.2533x; 1.0201x over previous
import functools

import jax
import jax.numpy as jnp
from jax import lax
from jax.experimental import pallas as pl
from jax.experimental.pallas import tpu as pltpu
from jax.experimental.pallas import tpu_sc as plsc

N = 1_000_000
K = 16_384
L = 16
NC, NS = 2, 16
NW = NC * NS
SHARD = 31_256
LAST_BASE = N - SHARD
CHUNKS = K // L


_mesh = plsc.VectorSubcoreMesh(core_axis_name="c", subcore_axis_name="s")


@functools.partial(
    pl.kernel,
    mesh=_mesh,
    out_type=jax.ShapeDtypeStruct((N,), jnp.float32),
    scratch_types=[
        pltpu.VMEM((SHARD,), jnp.float32),
        pltpu.VMEM((K,), jnp.int32),
        pltpu.VMEM((K,), jnp.float32),
        pltpu.VMEM_SHARED((K,), jnp.int32),
        pltpu.VMEM_SHARED((K,), jnp.float32),
        pltpu.SemaphoreType.DMA,
        pltpu.SemaphoreType.DMA,
    ],
    compiler_params=pltpu.CompilerParams(needs_layout_passes=False),
)
def _scatter_set(in_hbm, idx_hbm, val_hbm, out_hbm, shard_v, idx_v, val_v,
                 idx_sh, val_sh, sem, sem2):
    cid = lax.axis_index("c")
    sid = lax.axis_index("s")
    wid = sid * NC + cid
    base = jnp.where(wid == NW - 1, LAST_BASE, wid * SHARD)

    shard_cpy = pltpu.async_copy(in_hbm.at[pl.ds(base, SHARD)], shard_v, sem)

    kslice = K // NS
    off = sid * kslice
    sicpy = pltpu.async_copy(idx_hbm.at[pl.ds(off, kslice)],
                             idx_sh.at[pl.ds(off, kslice)], sem2)
    svcpy = pltpu.async_copy(val_hbm.at[pl.ds(off, kslice)],
                             val_sh.at[pl.ds(off, kslice)], sem2)
    sicpy.wait()
    svcpy.wait()
    plsc.subcore_barrier()
    icpy = pltpu.async_copy(idx_sh, idx_v, sem2)
    vcpy = pltpu.async_copy(val_sh, val_v, sem2)
    icpy.wait()
    vcpy.wait()
    shard_cpy.wait()

    U = 16

    def body(g, carry):
        s0 = g * (L * U)
        rels, vvs, ms = [], [], []
        for k in range(U):
            iv = idx_v[pl.ds(s0 + k * L, L)]
            vvs.append(val_v[pl.ds(s0 + k * L, L)])
            rel = iv - base
            rels.append(rel)
            ms.append(plsc.bitcast(rel, jnp.uint32) < jnp.uint32(SHARD))
        lasts = [plsc.scan_count(rels[k], ms[k])[1] for k in range(U)]
        for k in range(U):
            plsc.store_scatter(shard_v, [rels[k]], vvs[k],
                               mask=lasts[k] & ms[k])
        return carry

    lax.fori_loop(0, CHUNKS // U, body, 0, unroll=1)

    pltpu.sync_copy(shard_v, out_hbm.at[pl.ds(base, SHARD)])


def kernel(input, index, value):
    return _scatter_set(input, index.astype(jnp.int32), value)
